# baseline (speedup 1.0000x reference)
import functools

import jax
import jax.numpy as jnp
from jax import lax
from jax.experimental import pallas as pl
from jax.experimental.pallas import tpu as pltpu
from jax.experimental.pallas import tpu_sc as plsc

B, L, V, H, T = 1024, 200, 100000, 128, 32
N = B * L
CHUNK = 64
RB = 2048
EPS = 1e-12

_NC, _NS = 2, 16
_NW = _NC * _NS

_NSPLIT = 4
_NROWS = N // _NSPLIT
_NBH = _NROWS // RB
_NBUF = 5


@functools.cache
def _sc_gather_kernel(n_rows):
    rpw = n_rows // _NW
    cpw = rpw // CHUNK
    ngrp = cpw // _NBUF
    assert cpw % _NBUF == 0
    mesh = plsc.VectorSubcoreMesh(core_axis_name="c", subcore_axis_name="s")

    @functools.partial(
        pl.kernel,
        mesh=mesh,
        out_type=jax.ShapeDtypeStruct((n_rows, H), jnp.float32),
        scratch_types=[
            pltpu.VMEM((cpw, CHUNK), jnp.int32),
        ] + [pltpu.VMEM((CHUNK, H), jnp.float32) for _ in range(_NBUF)]
          + [pltpu.SemaphoreType.DMA for _ in range(2 * _NBUF)],
    )
    def _sc_gather(word_hbm, idx_hbm, out_hbm, idx_v, *bufs_sems):
        bufs = bufs_sems[:_NBUF]
        gsem = bufs_sems[_NBUF:2 * _NBUF]
        wsem = bufs_sems[2 * _NBUF:]
        wid = lax.axis_index("s") * _NC + lax.axis_index("c")
        pltpu.sync_copy(idx_hbm.at[wid], idx_v)
        row_base = wid * rpw

        for b in range(_NBUF):
            pltpu.async_copy(word_hbm.at[idx_v.at[b]], bufs[b], gsem[b])

        def _group(p, carry):
            for b in range(_NBUF):
                c = p * _NBUF + b
                pltpu.make_async_copy(word_hbm.at[idx_v.at[c]], bufs[b],
                                      gsem[b]).wait()
                pltpu.async_copy(
                    bufs[b], out_hbm.at[pl.ds(row_base + c * CHUNK, CHUNK)],
                    wsem[b])
            for b in range(_NBUF):
                @pl.when(p + 1 < ngrp)
                def _():
                    c2 = (p + 1) * _NBUF + b
                    pltpu.make_async_copy(
                        bufs[b], out_hbm.at[pl.ds(row_base, CHUNK)],
                        wsem[b]).wait()
                    pltpu.async_copy(word_hbm.at[idx_v.at[c2]], bufs[b],
                                     gsem[b])
            return carry

        lax.fori_loop(0, ngrp, _group, 0)
        for b in range(_NBUF):
            pltpu.make_async_copy(bufs[b], out_hbm.at[pl.ds(row_base, CHUNK)],
                                  wsem[b]).wait()

    return _sc_gather


def _fast_sin(x):
    f32 = jnp.float32
    k = jnp.round(x * f32(0.15915494309189535))
    r = x - k * f32(6.2831854820251465)
    r2 = r * r
    p = f32(2.7557319223985893e-06) + r2 * f32(-2.505210838544172e-08)
    p = f32(-0.0001984126984126984) + r2 * p
    p = f32(0.008333333333333333) + r2 * p
    p = f32(-0.16666666666666666) + r2 * p
    return r + r * r2 * p


_DN0 = (((0,), (0,)), ((), ()))


def _tc_body(code_ref, sc_ref, fw_ref, fphi_ref,
             w_ref, b_ref, ts_ref, ord_ref, g_ref, beta_ref,
             out_ref):
    f32 = jnp.float32
    sc = sc_ref[...]
    daT = sc[0:2]
    daT_hi = daT.astype(jnp.bfloat16).astype(f32)
    daT_lo = daT - daT_hi
    fw = fw_ref[...]
    fw_hi = fw.astype(jnp.bfloat16).astype(f32)
    fw_lo = fw - fw_hi
    arg = (lax.dot_general(daT_hi, fw_hi, _DN0, preferred_element_type=f32)
           + lax.dot_general(daT_lo, fw_hi, _DN0, preferred_element_type=f32)
           + lax.dot_general(daT_hi, fw_lo, _DN0, preferred_element_type=f32))
    feats = _fast_sin(arg + fphi_ref[...])
    w = w_ref[...]
    acc = jnp.dot(code_ref[...], w[:H], preferred_element_type=f32)
    acc += jnp.dot(feats, w[H:], preferred_element_type=f32)
    tok = jnp.tanh(acc + b_ref[...])
    i32 = jnp.int32
    oh_ts = (sc[2:3, :].astype(i32)
             == lax.broadcasted_iota(i32, (32, RB), 0)).astype(f32)
    oh_o = (sc[3:4, :].astype(i32)
            == lax.broadcasted_iota(i32, (512, RB), 0)).astype(jnp.bfloat16)
    tok += lax.dot_general(oh_ts, ts_ref[...], _DN0, preferred_element_type=f32)
    tok += lax.dot_general(oh_o, ord_ref[...], _DN0, preferred_element_type=f32)
    ones_m = jnp.full((H, H), 1.0 / H, f32)
    mu = jnp.dot(tok, ones_m, preferred_element_type=f32)
    xc = tok - mu
    var = jnp.dot(xc * xc, ones_m, preferred_element_type=f32)
    out_ref[...] = xc * lax.rsqrt(var + EPS) * g_ref[...] + beta_ref[...]


def _row_spec(cols):
    return pl.BlockSpec((RB, cols), lambda i: (i, 0))


def _full_spec(shape):
    return pl.BlockSpec(shape, lambda i: (0, 0))


def _tc_body_alias(prev_ref, *refs):
    del prev_ref
    _tc_body(*refs)


@functools.cache
def _tc_call_half(h):
    in_specs = [
        _row_spec(H),
        pl.BlockSpec((8, RB), lambda i: (0, i + h * _NBH)),
        _full_spec((2, 2 * T)), _full_spec((1, 2 * T)),
        _full_spec((H + 2 * T, H)), _full_spec((1, H)),
        _full_spec((32, H)), _full_spec((512, H)),
        _full_spec((1, H)), _full_spec((1, H)),
    ]
    out_specs = pl.BlockSpec((RB, H), lambda i: (i + h * _NBH, 0))
    out_shape = jax.ShapeDtypeStruct((N, H), jnp.float32)
    if h == 0:
        return pl.pallas_call(
            _tc_body, grid=(_NBH,), in_specs=in_specs,
            out_specs=out_specs, out_shape=out_shape)
    return pl.pallas_call(
        _tc_body_alias, grid=(_NBH,),
        in_specs=[pl.BlockSpec(memory_space=pl.ANY)] + in_specs,
        out_specs=out_specs, out_shape=out_shape,
        input_output_aliases={0: 0})


def kernel(input_ids, token_type_ids, time_stamps, ages, visit_orders,
           visit_segments, word_emb, type_emb, order_emb, seg_emb,
           time_w, time_phi, age_w, age_phi, proj_W, proj_b,
           ln_gamma, ln_beta):
    f32 = jnp.float32
    word_f = word_emb.astype(f32)
    cpw = _NROWS // _NW // CHUNK
    ids = input_ids.astype(jnp.int32).reshape(_NSPLIT, _NW, cpw, CHUNK)
    codes = [_sc_gather_kernel(_NROWS)(word_f, ids[h])
             for h in range(_NSPLIT)]

    ts = time_stamps.astype(f32)
    deltas = jnp.concatenate([ts[:, :1] * 0.0, ts[:, 1:] - ts[:, :-1]], axis=-1)
    tt_i = token_type_ids.astype(jnp.int32).reshape(N)
    vs_i = visit_segments.astype(jnp.int32).reshape(N)
    d_f = deltas.reshape(N)
    a_f = ages.astype(f32).reshape(N)
    zrow = jnp.zeros((N,), f32)
    scal = jnp.stack(
        [d_f, a_f,
         (tt_i * 3 + vs_i).astype(f32),
         visit_orders.astype(f32).reshape(N),
         zrow, zrow, zrow, zrow], axis=0)
    zero = jnp.zeros((1, T), f32)
    fw = jnp.concatenate(
        [jnp.concatenate([time_w.astype(f32), zero], axis=1),
         jnp.concatenate([zero, age_w.astype(f32)], axis=1)], axis=0)
    fphi = jnp.concatenate([time_phi.astype(f32), age_phi.astype(f32)], axis=1)

    tables = (
        fw, fphi,
        proj_W.astype(f32), proj_b.astype(f32).reshape(1, H),
        jnp.pad((type_emb.astype(f32)[:, None, :]
                 + seg_emb.astype(f32)[None, :, :]).reshape(30, H),
                ((0, 2), (0, 0))),
        order_emb.astype(jnp.bfloat16),
        ln_gamma.astype(f32).reshape(1, H),
        ln_beta.astype(f32).reshape(1, H),
    )
    out2d = _tc_call_half(0)(codes[0], scal, *tables)
    for h in range(1, _NSPLIT):
        out2d = _tc_call_half(h)(out2d, codes[h], scal, *tables)
    return out2d.reshape(B, L, H)


# speedup vs baseline: 1.0209x; 1.0209x over previous
import functools

import jax
import jax.numpy as jnp
from jax import lax
from jax.experimental import pallas as pl
from jax.experimental.pallas import tpu as pltpu
from jax.experimental.pallas import tpu_sc as plsc

B, L, V, H, T = 1024, 200, 100000, 128, 32
N = B * L
CHUNK = 128
RB = 2048
EPS = 1e-12

_NC, _NS = 2, 16
_NW = _NC * _NS

_NSPLIT = 2
_NROWS = N // _NSPLIT
_NBH = _NROWS // RB
_NBUF = 5


@functools.cache
def _sc_gather_kernel(n_rows):
    rpw = n_rows // _NW
    cpw = rpw // CHUNK
    ngrp = cpw // _NBUF
    assert cpw % _NBUF == 0
    mesh = plsc.VectorSubcoreMesh(core_axis_name="c", subcore_axis_name="s")

    @functools.partial(
        pl.kernel,
        mesh=mesh,
        out_type=jax.ShapeDtypeStruct((n_rows, H), jnp.float32),
        scratch_types=[
            pltpu.VMEM((cpw, CHUNK), jnp.int32),
        ] + [pltpu.VMEM((CHUNK, H), jnp.float32) for _ in range(_NBUF)]
          + [pltpu.SemaphoreType.DMA for _ in range(2 * _NBUF)],
    )
    def _sc_gather(word_hbm, idx_hbm, out_hbm, idx_v, *bufs_sems):
        bufs = bufs_sems[:_NBUF]
        gsem = bufs_sems[_NBUF:2 * _NBUF]
        wsem = bufs_sems[2 * _NBUF:]
        wid = lax.axis_index("s") * _NC + lax.axis_index("c")
        pltpu.sync_copy(idx_hbm.at[wid], idx_v)
        row_base = wid * rpw

        for b in range(_NBUF):
            pltpu.async_copy(word_hbm.at[idx_v.at[b]], bufs[b], gsem[b])

        def _group(p, carry):
            for b in range(_NBUF):
                c = p * _NBUF + b
                pltpu.make_async_copy(word_hbm.at[idx_v.at[c]], bufs[b],
                                      gsem[b]).wait()
                pltpu.async_copy(
                    bufs[b], out_hbm.at[pl.ds(row_base + c * CHUNK, CHUNK)],
                    wsem[b])
            for b in range(_NBUF):
                @pl.when(p + 1 < ngrp)
                def _():
                    c2 = (p + 1) * _NBUF + b
                    pltpu.make_async_copy(
                        bufs[b], out_hbm.at[pl.ds(row_base, CHUNK)],
                        wsem[b]).wait()
                    pltpu.async_copy(word_hbm.at[idx_v.at[c2]], bufs[b],
                                     gsem[b])
            return carry

        lax.fori_loop(0, ngrp, _group, 0)
        for b in range(_NBUF):
            pltpu.make_async_copy(bufs[b], out_hbm.at[pl.ds(row_base, CHUNK)],
                                  wsem[b]).wait()

    return _sc_gather


def _fast_sin(x):
    f32 = jnp.float32
    k = jnp.round(x * f32(0.15915494309189535))
    r = x - k * f32(6.2831854820251465)
    r2 = r * r
    p = f32(2.7557319223985893e-06) + r2 * f32(-2.505210838544172e-08)
    p = f32(-0.0001984126984126984) + r2 * p
    p = f32(0.008333333333333333) + r2 * p
    p = f32(-0.16666666666666666) + r2 * p
    return r + r * r2 * p


_DN0 = (((0,), (0,)), ((), ()))


def _tc_body(code_ref, sc_ref, fw_ref, fphi_ref,
             w_ref, b_ref, ts_ref, ord_ref, g_ref, beta_ref,
             out_ref):
    f32 = jnp.float32
    sc = sc_ref[...]
    daT = sc[0:2]
    daT_hi = daT.astype(jnp.bfloat16).astype(f32)
    daT_lo = daT - daT_hi
    fw = fw_ref[...]
    fw_hi = fw.astype(jnp.bfloat16).astype(f32)
    fw_lo = fw - fw_hi
    arg = (lax.dot_general(daT_hi, fw_hi, _DN0, preferred_element_type=f32)
           + lax.dot_general(daT_lo, fw_hi, _DN0, preferred_element_type=f32)
           + lax.dot_general(daT_hi, fw_lo, _DN0, preferred_element_type=f32))
    feats = _fast_sin(arg + fphi_ref[...])
    w = w_ref[...]
    acc = jnp.dot(code_ref[...], w[:H], preferred_element_type=f32)
    acc += jnp.dot(feats, w[H:], preferred_element_type=f32)
    tok = jnp.tanh(acc + b_ref[...])
    i32 = jnp.int32
    oh_ts = (sc[2:3, :].astype(i32)
             == lax.broadcasted_iota(i32, (32, RB), 0)).astype(f32)
    oh_o = (sc[3:4, :].astype(i32)
            == lax.broadcasted_iota(i32, (512, RB), 0)).astype(jnp.bfloat16)
    tok += lax.dot_general(oh_ts, ts_ref[...], _DN0, preferred_element_type=f32)
    tok += lax.dot_general(oh_o, ord_ref[...], _DN0, preferred_element_type=f32)
    ones_m = jnp.full((H, H), 1.0 / H, f32)
    mu = jnp.dot(tok, ones_m, preferred_element_type=f32)
    xc = tok - mu
    var = jnp.dot(xc * xc, ones_m, preferred_element_type=f32)
    out_ref[...] = xc * lax.rsqrt(var + EPS) * g_ref[...] + beta_ref[...]


def _row_spec(cols):
    return pl.BlockSpec((RB, cols), lambda i: (i, 0))


def _full_spec(shape):
    return pl.BlockSpec(shape, lambda i: (0, 0))


def _tc_body_alias(prev_ref, *refs):
    del prev_ref
    _tc_body(*refs)


@functools.cache
def _tc_call_half(h):
    in_specs = [
        _row_spec(H),
        pl.BlockSpec((8, RB), lambda i: (0, i + h * _NBH)),
        _full_spec((2, 2 * T)), _full_spec((1, 2 * T)),
        _full_spec((H + 2 * T, H)), _full_spec((1, H)),
        _full_spec((32, H)), _full_spec((512, H)),
        _full_spec((1, H)), _full_spec((1, H)),
    ]
    out_specs = pl.BlockSpec((RB, H), lambda i: (i + h * _NBH, 0))
    out_shape = jax.ShapeDtypeStruct((N, H), jnp.float32)
    if h == 0:
        return pl.pallas_call(
            _tc_body, grid=(_NBH,), in_specs=in_specs,
            out_specs=out_specs, out_shape=out_shape)
    return pl.pallas_call(
        _tc_body_alias, grid=(_NBH,),
        in_specs=[pl.BlockSpec(memory_space=pl.ANY)] + in_specs,
        out_specs=out_specs, out_shape=out_shape,
        input_output_aliases={0: 0})


def kernel(input_ids, token_type_ids, time_stamps, ages, visit_orders,
           visit_segments, word_emb, type_emb, order_emb, seg_emb,
           time_w, time_phi, age_w, age_phi, proj_W, proj_b,
           ln_gamma, ln_beta):
    f32 = jnp.float32
    word_f = word_emb.astype(f32)
    cpw = _NROWS // _NW // CHUNK
    ids = input_ids.astype(jnp.int32).reshape(_NSPLIT, _NW, cpw, CHUNK)
    codes = [_sc_gather_kernel(_NROWS)(word_f, ids[h])
             for h in range(_NSPLIT)]

    ts = time_stamps.astype(f32)
    deltas = jnp.concatenate([ts[:, :1] * 0.0, ts[:, 1:] - ts[:, :-1]], axis=-1)
    tt_i = token_type_ids.astype(jnp.int32).reshape(N)
    vs_i = visit_segments.astype(jnp.int32).reshape(N)
    d_f = deltas.reshape(N)
    a_f = ages.astype(f32).reshape(N)
    zrow = jnp.zeros((N,), f32)
    scal = jnp.stack(
        [d_f, a_f,
         (tt_i * 3 + vs_i).astype(f32),
         visit_orders.astype(f32).reshape(N),
         zrow, zrow, zrow, zrow], axis=0)
    zero = jnp.zeros((1, T), f32)
    fw = jnp.concatenate(
        [jnp.concatenate([time_w.astype(f32), zero], axis=1),
         jnp.concatenate([zero, age_w.astype(f32)], axis=1)], axis=0)
    fphi = jnp.concatenate([time_phi.astype(f32), age_phi.astype(f32)], axis=1)

    tables = (
        fw, fphi,
        proj_W.astype(f32), proj_b.astype(f32).reshape(1, H),
        jnp.pad((type_emb.astype(f32)[:, None, :]
                 + seg_emb.astype(f32)[None, :, :]).reshape(30, H),
                ((0, 2), (0, 0))),
        order_emb.astype(jnp.bfloat16),
        ln_gamma.astype(f32).reshape(1, H),
        ln_beta.astype(f32).reshape(1, H),
    )
    out2d = _tc_call_half(0)(codes[0], scal, *tables)
    out2d = _tc_call_half(1)(out2d, codes[1], scal, *tables)
    return out2d.reshape(B, L, H)
